# baseline (speedup 1.0000x reference)
import functools

import jax
import jax.numpy as jnp
from jax import lax
from jax.experimental import pallas as pl
from jax.experimental.pallas import tpu as pltpu
from jax.experimental.pallas import tpu_sc as plsc

_K = 1024
_C = 512
_BM = 512
_KT = 256


def _tc_argmin_body(z_ref, e_ref, s2_ref, idx_ref):
    zb = z_ref[...]
    s1 = jnp.sum(zb * zb, axis=1, keepdims=True)
    tiota = lax.broadcasted_iota(jnp.int32, (_BM, _KT), 1)
    vmin = jnp.full((_BM, _KT), jnp.inf, jnp.float32)
    varg = jnp.zeros((_BM, _KT), jnp.int32)
    for t in range(_K // _KT):
        et = e_ref[t * _KT:(t + 1) * _KT, :]
        mm = lax.dot_general(zb, et, (((1,), (1,)), ((), ())),
                             preferred_element_type=jnp.float32)
        d = (s1 - 2.0 * mm) + s2_ref[0, t * _KT:(t + 1) * _KT]
        cmp = d < vmin
        vmin = jnp.where(cmp, d, vmin)
        varg = jnp.where(cmp, tiota + t * _KT, varg)
    m = jnp.min(vmin, axis=1, keepdims=True)
    idx = jnp.min(jnp.where(vmin == m, varg, _K), axis=1)
    idx_ref[0, 0, :] = idx


def _tc_argmin(z_flat, e, s2t, ci, nblk):
    base = ci * nblk
    out = pl.pallas_call(
        _tc_argmin_body,
        grid=(nblk,),
        in_specs=[
            pl.BlockSpec((_BM, _C), lambda i: (base + i, 0)),
            pl.BlockSpec((_K, _C), lambda i: (0, 0)),
            pl.BlockSpec((1, _K), lambda i: (0, 0)),
        ],
        out_specs=pl.BlockSpec((1, 1, _BM), lambda i: (i, 0, 0)),
        out_shape=jax.ShapeDtypeStruct((nblk, 1, _BM), jnp.int32),
    )(z_flat, e, s2t)
    return out.reshape(nblk * _BM)


_NW = 32
_CH = 64


def _sc_gather_body(table_hbm, idx_hbm, out_hbm, idx_v, rows_v, sem0, sem1):
    wid = lax.axis_index("s") * 2 + lax.axis_index("c")
    b_per_w = idx_v.shape[0]
    nch = b_per_w // _CH
    base = wid * b_per_w
    pltpu.sync_copy(idx_hbm.at[pl.ds(base, b_per_w)], idx_v)
    sems = (sem0, sem1)

    def start(c):
        return pltpu.async_copy(
            table_hbm.at[idx_v.at[pl.ds(c * _CH, _CH)]],
            rows_v.at[c % 2], sems[c % 2])

    cp = start(0)
    for c in range(nch):
        nxt = start(c + 1) if c + 1 < nch else None
        cp.wait()
        pltpu.sync_copy(rows_v.at[c % 2],
                        out_hbm.at[pl.ds(base + c * _CH, _CH)])
        cp = nxt


def _sc_gather(table, idx):
    n = idx.shape[0]
    b_per_w = n // _NW
    mesh = plsc.VectorSubcoreMesh(core_axis_name="c", subcore_axis_name="s")
    f = functools.partial(
        pl.kernel,
        out_type=jax.ShapeDtypeStruct((n, _C), jnp.float32),
        mesh=mesh,
        scratch_types=[
            pltpu.VMEM((b_per_w,), jnp.int32),
            pltpu.VMEM((2, _CH, _C), jnp.float32),
            pltpu.SemaphoreType.DMA,
            pltpu.SemaphoreType.DMA,
        ],
    )(_sc_gather_body)
    return f(table, idx)


def _tc_assemble_body(g_ref, o_ref):
    o_ref[0, :, :] = g_ref[...]


def _tc_assemble_body_acc(g_ref, buf_ref, o_ref):
    del buf_ref
    o_ref[0] = g_ref[...]


def _tc_assemble_chunk(gd_c, buf, ci, bc, nb):
    base = ci * bc
    in_specs = [pl.BlockSpec((_C, _C), lambda b, s: (b * 2 + s, 0))]
    if buf is None:
        args = (gd_c,)
        body = _tc_assemble_body
        aliases = {}
    else:
        args = (gd_c, buf)
        body = _tc_assemble_body_acc
        in_specs.append(pl.BlockSpec(memory_space=pl.ANY))
        aliases = {1: 0}
    return pl.pallas_call(
        body,
        grid=(bc, 2),
        in_specs=in_specs,
        out_specs=pl.BlockSpec((1, _C, _C), lambda b, s: (base + b, s, 0)),
        out_shape=jax.ShapeDtypeStruct((nb, 2 * _C, _C), jnp.float32),
        input_output_aliases=aliases,
    )(*args)


_NCHUNK = 4


def kernel(z, embedding_weight):
    B, C, H, W = z.shape
    z_flat = jnp.transpose(z, (0, 2, 3, 1)).reshape(-1, C)
    s2t = jnp.sum(embedding_weight ** 2, axis=1, keepdims=True).T
    bc = B // _NCHUNK
    nblk = bc * H * W // _BM
    buf = None
    idxs = []
    for ci in range(_NCHUNK):
        idx_c = _tc_argmin(z_flat, embedding_weight, s2t, ci, nblk)
        g_c = _sc_gather(embedding_weight, idx_c)
        buf = _tc_assemble_chunk(g_c, buf, ci, bc, B)
        idxs.append(idx_c.reshape(bc, -1))
    quantized = buf.reshape(B, C, H, W)
    idx = jnp.concatenate(idxs, axis=0)
    return (quantized, idx)


# speedup vs baseline: 1.0927x; 1.0927x over previous
import functools

import jax
import jax.numpy as jnp
from jax import lax
from jax.experimental import pallas as pl
from jax.experimental.pallas import tpu as pltpu
from jax.experimental.pallas import tpu_sc as plsc

_K = 1024
_C = 512
_BM = 512
_KT = 256


def _tc_argmin_body(z_ref, e_ref, s2_ref, idx_ref):
    zb = z_ref[...]
    s1 = jnp.sum(zb * zb, axis=1, keepdims=True)
    tiota = lax.broadcasted_iota(jnp.int32, (_BM, _KT), 1)
    vmin = jnp.full((_BM, _KT), jnp.inf, jnp.float32)
    varg = jnp.zeros((_BM, _KT), jnp.int32)
    for t in range(_K // _KT):
        et = e_ref[t * _KT:(t + 1) * _KT, :]
        mm = lax.dot_general(zb, et, (((1,), (1,)), ((), ())),
                             preferred_element_type=jnp.float32)
        d = (s1 - 2.0 * mm) + s2_ref[0, t * _KT:(t + 1) * _KT]
        cmp = d < vmin
        vmin = jnp.where(cmp, d, vmin)
        varg = jnp.where(cmp, tiota + t * _KT, varg)
    m = jnp.min(vmin, axis=1, keepdims=True)
    idx = jnp.min(jnp.where(vmin == m, varg, _K), axis=1)
    idx_ref[0, 0, :] = idx


def _tc_argmin(z_flat, e, s2t, ci, nblk):
    base = ci * nblk
    out = pl.pallas_call(
        _tc_argmin_body,
        grid=(nblk,),
        in_specs=[
            pl.BlockSpec((_BM, _C), lambda i: (base + i, 0)),
            pl.BlockSpec((_K, _C), lambda i: (0, 0)),
            pl.BlockSpec((1, _K), lambda i: (0, 0)),
        ],
        out_specs=pl.BlockSpec((1, 1, _BM), lambda i: (i, 0, 0)),
        out_shape=jax.ShapeDtypeStruct((nblk, 1, _BM), jnp.int32),
    )(z_flat, e, s2t)
    return out.reshape(nblk * _BM)


_NW = 32
_CH = 64


def _sc_gather_body(table_hbm, idx_hbm, out_hbm, idx_v, rows_v, sem0, sem1):
    wid = lax.axis_index("s") * 2 + lax.axis_index("c")
    b_per_w = idx_v.shape[0]
    nch = b_per_w // _CH
    base = wid * b_per_w
    pltpu.sync_copy(idx_hbm.at[pl.ds(base, b_per_w)], idx_v)
    sems = (sem0, sem1)

    def start(c):
        return pltpu.async_copy(
            table_hbm.at[idx_v.at[pl.ds(c * _CH, _CH)]],
            rows_v.at[c % 2], sems[c % 2])

    cp = start(0)
    for c in range(nch):
        nxt = start(c + 1) if c + 1 < nch else None
        cp.wait()
        pltpu.sync_copy(rows_v.at[c % 2],
                        out_hbm.at[pl.ds(base + c * _CH, _CH)])
        cp = nxt


def _sc_gather(table, idx):
    n = idx.shape[0]
    b_per_w = n // _NW
    mesh = plsc.VectorSubcoreMesh(core_axis_name="c", subcore_axis_name="s")
    f = functools.partial(
        pl.kernel,
        out_type=jax.ShapeDtypeStruct((n, _C), jnp.float32),
        mesh=mesh,
        scratch_types=[
            pltpu.VMEM((b_per_w,), jnp.int32),
            pltpu.VMEM((2, _CH, _C), jnp.float32),
            pltpu.SemaphoreType.DMA,
            pltpu.SemaphoreType.DMA,
        ],
    )(_sc_gather_body)
    return f(table, idx)


def kernel(z, embedding_weight):
    B, C, H, W = z.shape
    z_flat = jnp.transpose(z, (0, 2, 3, 1)).reshape(-1, C)
    s2t = jnp.sum(embedding_weight ** 2, axis=1, keepdims=True).T
    idx = _tc_argmin(z_flat, embedding_weight, s2t, 0, B * H * W // _BM)
    g = _sc_gather(embedding_weight, idx)
    quantized = g.reshape(B, C, H, W)
    return (quantized, idx.reshape(B, -1))
